# baseline (speedup 1.0000x reference)
import functools

import jax
import jax.numpy as jnp
from jax import lax
from jax.experimental import pallas as pl
from jax.experimental.pallas import tpu as pltpu
from jax.experimental.pallas import tpu_sc as plsc

N = 10000
E = 320000
D = 128
G = 16
NP = 10240
NC = 2
NS = 16
NW = NC * NS
CHUNK = 128
CPT = 80
BLK = 8
NB = CPT // BLK
EPT = CPT * CHUNK
EP = NW * EPT
RPT = NP // NS

def _deg_body(dst_hbm, out_hbm, dstb, ones_v, zbuf, degs):
    c = lax.axis_index("c")
    s = lax.axis_index("s")
    gid = c * NS + s
    ones16 = jnp.ones((16,), jnp.float32)
    for k in range(CHUNK // 16):
        ones_v[pl.ds(k * 16, 16)] = ones16

    @pl.when(s == 0)
    def _():
        zeros16 = jnp.zeros((16,), jnp.float32)
        zn = zbuf.shape[0]

        def zbody(i, carry):
            zbuf[pl.ds(i * 16, 16)] = zeros16
            return carry

        lax.fori_loop(0, zn // 16, zbody, 0)
        for i in range(NP // zn):
            pltpu.sync_copy(zbuf, degs.at[pl.ds(i * zn, zn)])

    plsc.subcore_barrier()

    def bbody(b, carry):
        pltpu.sync_copy(dst_hbm.at[gid].at[pl.ds(b * BLK, BLK)], dstb)

        def abody(j, carry2):
            pltpu.sync_copy(ones_v, degs.at[dstb.at[j]], add=True)
            return carry2

        lax.fori_loop(0, BLK, abody, 0)
        return carry

    lax.fori_loop(0, NB, bbody, 0)
    plsc.subcore_barrier()

    @pl.when(s == 0)
    def _():
        zn = zbuf.shape[0]
        for i in range(NP // zn):
            pltpu.sync_copy(degs.at[pl.ds(i * zn, zn)], zbuf)
            pltpu.sync_copy(zbuf, out_hbm.at[c].at[pl.ds(i * zn, zn)])


def _scatter_body(hs_hbm, src_hbm, dst_hbm, out_hbm,
                  srcb, dstb, buf0, buf1, accs, sem0, sem1):
    c = lax.axis_index("c")
    s = lax.axis_index("s")
    gid = c * NS + s

    zeros16 = jnp.zeros((16,), jnp.float32)

    def zbody(r, carry):
        for k in range(D // 16):
            buf0[r, pl.ds(k * 16, 16)] = zeros16
        return carry

    lax.fori_loop(0, CHUNK, zbody, 0)
    base = s * RPT
    for i in range(RPT // CHUNK):
        pltpu.sync_copy(buf0, accs.at[pl.ds(base + i * CHUNK, CHUNK)])
    plsc.subcore_barrier()

    def bbody(b, carry):
        pltpu.sync_copy(src_hbm.at[gid].at[pl.ds(b * BLK, BLK)], srcb)
        pltpu.sync_copy(dst_hbm.at[gid].at[pl.ds(b * BLK, BLK)], dstb)
        pltpu.async_copy(hs_hbm.at[srcb.at[0]], buf0, sem0)

        def body(i, carry2):
            j0 = i * 2
            j1 = j0 + 1
            pltpu.async_copy(hs_hbm.at[srcb.at[j1]], buf1, sem1)
            pltpu.make_async_copy(hs_hbm.at[srcb.at[j0]], buf0, sem0).wait()
            pltpu.sync_copy(buf0, accs.at[dstb.at[j0]], add=True)

            @pl.when(i < BLK // 2 - 1)
            def _():
                pltpu.async_copy(hs_hbm.at[srcb.at[j0 + 2]], buf0, sem0)

            pltpu.make_async_copy(hs_hbm.at[srcb.at[j1]], buf1, sem1).wait()
            pltpu.sync_copy(buf1, accs.at[dstb.at[j1]], add=True)
            return carry2

        lax.fori_loop(0, BLK // 2, body, 0)
        return carry

    lax.fori_loop(0, NB, bbody, 0)
    plsc.subcore_barrier()

    for i in range(RPT // CHUNK):
        pltpu.sync_copy(accs.at[pl.ds(base + i * CHUNK, CHUNK)], buf0)
        pltpu.sync_copy(buf0, out_hbm.at[c].at[pl.ds(base + i * CHUNK, CHUNK)])


@functools.cache
def _sc_kernels():
    mesh = plsc.VectorSubcoreMesh(
        core_axis_name="c", subcore_axis_name="s",
        num_cores=NC, num_subcores=NS,
    )
    deg = pl.kernel(
        _deg_body,
        out_type=jax.ShapeDtypeStruct((NC, NP), jnp.float32),
        mesh=mesh,
        scratch_types=[
            pltpu.VMEM((BLK, CHUNK), jnp.int32),
            pltpu.VMEM((CHUNK,), jnp.float32),
            pltpu.VMEM((2048,), jnp.float32),
            pltpu.VMEM_SHARED((NP,), jnp.float32),
        ],
    )
    scatter = pl.kernel(
        _scatter_body,
        out_type=jax.ShapeDtypeStruct((NC, NP, D), jnp.float32),
        mesh=mesh,
        scratch_types=[
            pltpu.VMEM((BLK, CHUNK), jnp.int32),
            pltpu.VMEM((BLK, CHUNK), jnp.int32),
            pltpu.VMEM((CHUNK, D), jnp.float32),
            pltpu.VMEM((CHUNK, D), jnp.float32),
            pltpu.VMEM_SHARED((NP, D), jnp.float32),
            pltpu.SemaphoreType.DMA,
            pltpu.SemaphoreType.DMA,
        ],
    )
    return deg, scatter


def _prep_body(x_ref, dega_ref, degb_ref, w_ref, out_ref):
    deg = dega_ref[...] + degb_ref[...]
    dinv = lax.rsqrt(deg + 1.0)
    out_ref[...] = (
        jnp.dot(x_ref[...], w_ref[...], preferred_element_type=jnp.float32)
        * dinv
    )


_prep = pl.pallas_call(
    _prep_body, out_shape=jax.ShapeDtypeStruct((NP, D), jnp.float32)
)


def _bn_lrelu(a0, a1, hs, dega, degb, b, gm, bt):
    deg = dega + degb
    dinv = lax.rsqrt(deg + 1.0)
    z = dinv * (a0 + a1 + hs) + b
    zm = z[:N]
    mean = jnp.mean(zm, axis=0, keepdims=True)
    dev = zm - mean
    var = jnp.mean(dev * dev, axis=0, keepdims=True)
    h = dev * lax.rsqrt(var + 1e-5) * gm + bt
    return jnp.where(h >= 0, h, 0.01 * h), dinv


def _layer_body(a0_ref, a1_ref, hs_ref, dega_ref, degb_ref,
                b_ref, g_ref, bt_ref, w_ref, out_ref):
    h, dinv = _bn_lrelu(a0_ref[...], a1_ref[...], hs_ref[...],
                        dega_ref[...], degb_ref[...],
                        b_ref[...], g_ref[...], bt_ref[...])
    out_ref[pl.ds(0, N), :] = (
        jnp.dot(h, w_ref[...], preferred_element_type=jnp.float32) * dinv[:N]
    )
    out_ref[pl.ds(N, NP - N), :] = jnp.zeros((NP - N, D), jnp.float32)


_layer = pl.pallas_call(
    _layer_body, out_shape=jax.ShapeDtypeStruct((NP, D), jnp.float32)
)


def _final_body(a0_ref, a1_ref, hs_ref, dega_ref, degb_ref,
                b_ref, g_ref, bt_ref, batch_ref, lw_ref, lb_ref, out_ref):
    h, _ = _bn_lrelu(a0_ref[...], a1_ref[...], hs_ref[...],
                     dega_ref[...], degb_ref[...],
                     b_ref[...], g_ref[...], bt_ref[...])
    brow = batch_ref[...][:, :N]
    oh = (lax.broadcasted_iota(jnp.int32, (G, N), 0) == brow).astype(
        jnp.float32
    )
    sums = jnp.dot(oh, h, preferred_element_type=jnp.float32)
    counts = jnp.sum(oh, axis=1, keepdims=True)
    pooled = sums / jnp.maximum(counts, 1.0)
    out_ref[...] = (
        jnp.dot(pooled, lw_ref[...], preferred_element_type=jnp.float32)
        + lb_ref[...]
    )


_final = pl.pallas_call(
    _final_body, out_shape=jax.ShapeDtypeStruct((G, 1), jnp.float32)
)


def kernel(x, edge_index, batch, W1, b1, g1, bt1, W2, b2, g2, bt2,
           W3, b3, g3, bt3, lw, lb):
    src = edge_index[0]
    dst = edge_index[1]
    pad = jnp.full((EP - E,), N, jnp.int32)
    srcp = jnp.concatenate([src, pad]).reshape(NW, CPT, CHUNK)
    dstp = jnp.concatenate([dst, pad]).reshape(NW, CPT, CHUNK)
    x_pad = jnp.pad(x, ((0, NP - N), (0, 0)))
    batch_row = jnp.pad(batch, (0, NP - N), constant_values=G).reshape(1, NP)

    _deg_kernel, _scatter_kernel = _sc_kernels()
    degs = _deg_kernel(dstp)
    dega = degs[0].reshape(NP, 1)
    degb = degs[1].reshape(NP, 1)

    b1r, g1r, bt1r = b1.reshape(1, D), g1.reshape(1, D), bt1.reshape(1, D)
    b2r, g2r, bt2r = b2.reshape(1, D), g2.reshape(1, D), bt2.reshape(1, D)
    b3r, g3r, bt3r = b3.reshape(1, D), g3.reshape(1, D), bt3.reshape(1, D)

    hs = _prep(x_pad, dega, degb, W1)
    acc = _scatter_kernel(hs, srcp, dstp)
    hs = _layer(acc[0], acc[1], hs, dega, degb, b1r, g1r, bt1r, W2)
    acc = _scatter_kernel(hs, srcp, dstp)
    hs = _layer(acc[0], acc[1], hs, dega, degb, b2r, g2r, bt2r, W3)
    acc = _scatter_kernel(hs, srcp, dstp)
    return _final(acc[0], acc[1], hs, dega, degb, b3r, g3r, bt3r,
                  batch_row, lw, lb.reshape(1, 1))


# speedup vs baseline: 1.0536x; 1.0536x over previous
import functools

import jax
import jax.numpy as jnp
from jax import lax
from jax.experimental import pallas as pl
from jax.experimental.pallas import tpu as pltpu
from jax.experimental.pallas import tpu_sc as plsc

N = 10000
E = 320000
D = 128
G = 16
NP = 10240
NC = 2
NS = 16
NW = NC * NS
CHUNK = 64
CPT = 160
BLK = 16
NB = CPT // BLK
EPT = CPT * CHUNK
EP = NW * EPT
RPT = NP // NS

def _deg_body(dst_hbm, out_hbm, dstb, ones_v, zbuf, degs):
    c = lax.axis_index("c")
    s = lax.axis_index("s")
    gid = c * NS + s
    ones16 = jnp.ones((16,), jnp.float32)
    for k in range(CHUNK // 16):
        ones_v[pl.ds(k * 16, 16)] = ones16

    @pl.when(s == 0)
    def _():
        zeros16 = jnp.zeros((16,), jnp.float32)
        zn = zbuf.shape[0]

        def zbody(i, carry):
            zbuf[pl.ds(i * 16, 16)] = zeros16
            return carry

        lax.fori_loop(0, zn // 16, zbody, 0)
        for i in range(NP // zn):
            pltpu.sync_copy(zbuf, degs.at[pl.ds(i * zn, zn)])

    plsc.subcore_barrier()

    def bbody(b, carry):
        pltpu.sync_copy(dst_hbm.at[gid].at[pl.ds(b * BLK, BLK)], dstb)

        def abody(j, carry2):
            pltpu.sync_copy(ones_v, degs.at[dstb.at[j]], add=True)
            return carry2

        lax.fori_loop(0, BLK, abody, 0)
        return carry

    lax.fori_loop(0, NB, bbody, 0)
    plsc.subcore_barrier()

    @pl.when(s == 0)
    def _():
        zn = zbuf.shape[0]
        for i in range(NP // zn):
            pltpu.sync_copy(degs.at[pl.ds(i * zn, zn)], zbuf)
            pltpu.sync_copy(zbuf, out_hbm.at[c].at[pl.ds(i * zn, zn)])


def _scatter_body(hs_hbm, src_hbm, dst_hbm, out_hbm,
                  srcb, dstb, buf0, buf1, buf2, buf3, accs,
                  g0, g1, g2, g3, s0, s1, s2, s3):
    c = lax.axis_index("c")
    s = lax.axis_index("s")
    gid = c * NS + s
    bufs = (buf0, buf1, buf2, buf3)
    gsems = (g0, g1, g2, g3)
    ssems = (s0, s1, s2, s3)

    zeros16 = jnp.zeros((16,), jnp.float32)

    def zbody(r, carry):
        for k in range(D // 16):
            buf0[r, pl.ds(k * 16, 16)] = zeros16
        return carry

    lax.fori_loop(0, CHUNK, zbody, 0)
    base = s * RPT
    for i in range(RPT // CHUNK):
        pltpu.sync_copy(buf0, accs.at[pl.ds(base + i * CHUNK, CHUNK)])
    plsc.subcore_barrier()

    def bbody(b, carry):
        pltpu.sync_copy(src_hbm.at[gid].at[pl.ds(b * BLK, BLK)], srcb)
        pltpu.sync_copy(dst_hbm.at[gid].at[pl.ds(b * BLK, BLK)], dstb)
        gd = [None] * BLK
        sd = [None] * BLK
        for j in range(BLK):
            k = j % 4
            if j >= 4:
                sd[j - 4].wait()
            gd[j] = pltpu.async_copy(hs_hbm.at[srcb.at[j]], bufs[k], gsems[k])
            if j >= 2:
                p = j - 2
                gd[p].wait()
                sd[p] = pltpu.async_copy(
                    bufs[p % 4], accs.at[dstb.at[p]], ssems[p % 4], add=True
                )
        for p in (BLK - 2, BLK - 1):
            gd[p].wait()
            sd[p] = pltpu.async_copy(
                bufs[p % 4], accs.at[dstb.at[p]], ssems[p % 4], add=True
            )
        for p in range(BLK - 4, BLK):
            sd[p].wait()
        return carry

    lax.fori_loop(0, NB, bbody, 0)
    plsc.subcore_barrier()

    for i in range(RPT // CHUNK):
        pltpu.sync_copy(accs.at[pl.ds(base + i * CHUNK, CHUNK)], buf0)
        pltpu.sync_copy(buf0, out_hbm.at[c].at[pl.ds(base + i * CHUNK, CHUNK)])


@functools.cache
def _sc_kernels():
    mesh = plsc.VectorSubcoreMesh(
        core_axis_name="c", subcore_axis_name="s",
        num_cores=NC, num_subcores=NS,
    )
    deg = pl.kernel(
        _deg_body,
        out_type=jax.ShapeDtypeStruct((NC, NP), jnp.float32),
        mesh=mesh,
        scratch_types=[
            pltpu.VMEM((BLK, CHUNK), jnp.int32),
            pltpu.VMEM((CHUNK,), jnp.float32),
            pltpu.VMEM((2048,), jnp.float32),
            pltpu.VMEM_SHARED((NP,), jnp.float32),
        ],
    )
    scatter = pl.kernel(
        _scatter_body,
        out_type=jax.ShapeDtypeStruct((NC, NP, D), jnp.float32),
        mesh=mesh,
        scratch_types=[
            pltpu.VMEM((BLK, CHUNK), jnp.int32),
            pltpu.VMEM((BLK, CHUNK), jnp.int32),
            pltpu.VMEM((CHUNK, D), jnp.float32),
            pltpu.VMEM((CHUNK, D), jnp.float32),
            pltpu.VMEM((CHUNK, D), jnp.float32),
            pltpu.VMEM((CHUNK, D), jnp.float32),
            pltpu.VMEM_SHARED((NP, D), jnp.float32),
            pltpu.SemaphoreType.DMA,
            pltpu.SemaphoreType.DMA,
            pltpu.SemaphoreType.DMA,
            pltpu.SemaphoreType.DMA,
            pltpu.SemaphoreType.DMA,
            pltpu.SemaphoreType.DMA,
            pltpu.SemaphoreType.DMA,
            pltpu.SemaphoreType.DMA,
        ],
    )
    return deg, scatter


def _prep_body(x_ref, dega_ref, degb_ref, w_ref, out_ref):
    deg = dega_ref[...] + degb_ref[...]
    dinv = lax.rsqrt(deg + 1.0)
    out_ref[...] = (
        jnp.dot(x_ref[...], w_ref[...], preferred_element_type=jnp.float32)
        * dinv
    )


_prep = pl.pallas_call(
    _prep_body, out_shape=jax.ShapeDtypeStruct((NP, D), jnp.float32)
)


def _bn_lrelu(a0, a1, hs, dega, degb, b, gm, bt):
    deg = dega + degb
    dinv = lax.rsqrt(deg + 1.0)
    z = dinv * (a0 + a1 + hs) + b
    zm = z[:N]
    mean = jnp.mean(zm, axis=0, keepdims=True)
    dev = zm - mean
    var = jnp.mean(dev * dev, axis=0, keepdims=True)
    h = dev * lax.rsqrt(var + 1e-5) * gm + bt
    return jnp.where(h >= 0, h, 0.01 * h), dinv


def _layer_body(a0_ref, a1_ref, hs_ref, dega_ref, degb_ref,
                b_ref, g_ref, bt_ref, w_ref, out_ref):
    h, dinv = _bn_lrelu(a0_ref[...], a1_ref[...], hs_ref[...],
                        dega_ref[...], degb_ref[...],
                        b_ref[...], g_ref[...], bt_ref[...])
    out_ref[pl.ds(0, N), :] = (
        jnp.dot(h, w_ref[...], preferred_element_type=jnp.float32) * dinv[:N]
    )
    out_ref[pl.ds(N, NP - N), :] = jnp.zeros((NP - N, D), jnp.float32)


_layer = pl.pallas_call(
    _layer_body, out_shape=jax.ShapeDtypeStruct((NP, D), jnp.float32)
)


def _final_body(a0_ref, a1_ref, hs_ref, dega_ref, degb_ref,
                b_ref, g_ref, bt_ref, batch_ref, lw_ref, lb_ref, out_ref):
    h, _ = _bn_lrelu(a0_ref[...], a1_ref[...], hs_ref[...],
                     dega_ref[...], degb_ref[...],
                     b_ref[...], g_ref[...], bt_ref[...])
    brow = batch_ref[...][:, :N]
    oh = (lax.broadcasted_iota(jnp.int32, (G, N), 0) == brow).astype(
        jnp.float32
    )
    sums = jnp.dot(oh, h, preferred_element_type=jnp.float32)
    counts = jnp.sum(oh, axis=1, keepdims=True)
    pooled = sums / jnp.maximum(counts, 1.0)
    out_ref[...] = (
        jnp.dot(pooled, lw_ref[...], preferred_element_type=jnp.float32)
        + lb_ref[...]
    )


_final = pl.pallas_call(
    _final_body, out_shape=jax.ShapeDtypeStruct((G, 1), jnp.float32)
)


def kernel(x, edge_index, batch, W1, b1, g1, bt1, W2, b2, g2, bt2,
           W3, b3, g3, bt3, lw, lb):
    src = edge_index[0]
    dst = edge_index[1]
    pad = jnp.full((EP - E,), N, jnp.int32)
    srcp = jnp.concatenate([src, pad]).reshape(NW, CPT, CHUNK)
    dstp = jnp.concatenate([dst, pad]).reshape(NW, CPT, CHUNK)
    x_pad = jnp.pad(x, ((0, NP - N), (0, 0)))
    batch_row = jnp.pad(batch, (0, NP - N), constant_values=G).reshape(1, NP)

    _deg_kernel, _scatter_kernel = _sc_kernels()
    degs = _deg_kernel(dstp)
    dega = degs[0].reshape(NP, 1)
    degb = degs[1].reshape(NP, 1)

    b1r, g1r, bt1r = b1.reshape(1, D), g1.reshape(1, D), bt1.reshape(1, D)
    b2r, g2r, bt2r = b2.reshape(1, D), g2.reshape(1, D), bt2.reshape(1, D)
    b3r, g3r, bt3r = b3.reshape(1, D), g3.reshape(1, D), bt3.reshape(1, D)

    hs = _prep(x_pad, dega, degb, W1)
    acc = _scatter_kernel(hs, srcp, dstp)
    hs = _layer(acc[0], acc[1], hs, dega, degb, b1r, g1r, bt1r, W2)
    acc = _scatter_kernel(hs, srcp, dstp)
    hs = _layer(acc[0], acc[1], hs, dega, degb, b2r, g2r, bt2r, W3)
    acc = _scatter_kernel(hs, srcp, dstp)
    return _final(acc[0], acc[1], hs, dega, degb, b3r, g3r, bt3r,
                  batch_row, lw, lb.reshape(1, 1))


# speedup vs baseline: 1.9954x; 1.8939x over previous
import functools

import jax
import jax.numpy as jnp
from jax import lax
from jax.experimental import pallas as pl
from jax.experimental.pallas import tpu as pltpu
from jax.experimental.pallas import tpu_sc as plsc

N = 10000
E = 320000
D = 128
G = 16
NP = 10240
NC = 2
NS = 16
NW = NC * NS
HD = 64
CHUNK = 128
CPT = 80
BLK = 8
NB = CPT // BLK
EPT = CPT * CHUNK
EP = NW * EPT
RPT = NP // NS

def _deg_body(dst_hbm, out_hbm, dstb, ones_v, zbuf, degs):
    c = lax.axis_index("c")
    s = lax.axis_index("s")
    gid = c * NS + s
    ones16 = jnp.ones((16,), jnp.float32)
    for k in range(CHUNK // 16):
        ones_v[pl.ds(k * 16, 16)] = ones16

    @pl.when(s == 0)
    def _():
        zeros16 = jnp.zeros((16,), jnp.float32)
        zn = zbuf.shape[0]

        def zbody(i, carry):
            zbuf[pl.ds(i * 16, 16)] = zeros16
            return carry

        lax.fori_loop(0, zn // 16, zbody, 0)
        for i in range(NP // zn):
            pltpu.sync_copy(zbuf, degs.at[pl.ds(i * zn, zn)])

    plsc.subcore_barrier()

    def bbody(b, carry):
        pltpu.sync_copy(dst_hbm.at[gid].at[pl.ds(b * BLK, BLK)], dstb)

        def abody(j, carry2):
            pltpu.sync_copy(ones_v, degs.at[dstb.at[j]], add=True)
            return carry2

        lax.fori_loop(0, BLK, abody, 0)
        return carry

    lax.fori_loop(0, NB, bbody, 0)
    plsc.subcore_barrier()

    @pl.when(s == 0)
    def _():
        zn = zbuf.shape[0]
        for i in range(NP // zn):
            pltpu.sync_copy(degs.at[pl.ds(i * zn, zn)], zbuf)
            pltpu.sync_copy(zbuf, out_hbm.at[c].at[pl.ds(i * zn, zn)])


def _scatter_body(hsl_hbm, hsr_hbm, src_hbm, dst_hbm, out_hbm,
                  srcb, dstb, buf0, buf1, buf2, buf3, zf, hsm, accs,
                  g0, g1, g2, g3, s0, s1, s2, s3):
    c = lax.axis_index("c")
    s = lax.axis_index("s")
    gid = c * NS + s
    bufs = (buf0, buf1, buf2, buf3)
    gsems = (g0, g1, g2, g3)
    ssems = (s0, s1, s2, s3)
    base = s * RPT

    zeros16 = jnp.zeros((16,), jnp.float32)

    def zbody(r, carry):
        for k in range(HD // 16):
            zf[r, pl.ds(k * 16, 16)] = zeros16
        return carry

    lax.fori_loop(0, CHUNK, zbody, 0)

    for h, hs_hbm in enumerate((hsl_hbm, hsr_hbm)):
        pltpu.sync_copy(hs_hbm.at[pl.ds(base, RPT)], hsm.at[pl.ds(base, RPT)])
        for i in range(RPT // CHUNK):
            pltpu.sync_copy(zf, accs.at[pl.ds(base + i * CHUNK, CHUNK)])
        plsc.subcore_barrier()

        def bbody(b, carry):
            pltpu.sync_copy(src_hbm.at[gid].at[pl.ds(b * BLK, BLK)], srcb)
            pltpu.sync_copy(dst_hbm.at[gid].at[pl.ds(b * BLK, BLK)], dstb)
            gd = [None] * BLK
            sd = [None] * BLK
            for j in range(BLK):
                k = j % 4
                if j >= 4:
                    sd[j - 4].wait()
                gd[j] = pltpu.async_copy(hsm.at[srcb.at[j]], bufs[k], gsems[k])
                if j >= 2:
                    p = j - 2
                    gd[p].wait()
                    sd[p] = pltpu.async_copy(
                        bufs[p % 4], accs.at[dstb.at[p]], ssems[p % 4],
                        add=True,
                    )
            for p in (BLK - 2, BLK - 1):
                gd[p].wait()
                sd[p] = pltpu.async_copy(
                    bufs[p % 4], accs.at[dstb.at[p]], ssems[p % 4], add=True
                )
            for p in range(BLK - 4, BLK):
                sd[p].wait()
            return carry

        lax.fori_loop(0, NB, bbody, 0)
        plsc.subcore_barrier()

        for i in range(RPT // CHUNK):
            pltpu.sync_copy(accs.at[pl.ds(base + i * CHUNK, CHUNK)], buf0)
            pltpu.sync_copy(
                buf0, out_hbm.at[c].at[h].at[pl.ds(base + i * CHUNK, CHUNK)]
            )
        plsc.subcore_barrier()


@functools.cache
def _sc_kernels():
    mesh = plsc.VectorSubcoreMesh(
        core_axis_name="c", subcore_axis_name="s",
        num_cores=NC, num_subcores=NS,
    )
    deg = pl.kernel(
        _deg_body,
        out_type=jax.ShapeDtypeStruct((NC, NP), jnp.float32),
        mesh=mesh,
        scratch_types=[
            pltpu.VMEM((BLK, CHUNK), jnp.int32),
            pltpu.VMEM((CHUNK,), jnp.float32),
            pltpu.VMEM((2048,), jnp.float32),
            pltpu.VMEM_SHARED((NP,), jnp.float32),
        ],
    )
    scatter = pl.kernel(
        _scatter_body,
        out_type=jax.ShapeDtypeStruct((NC, 2, NP, HD), jnp.float32),
        mesh=mesh,
        compiler_params=pltpu.CompilerParams(use_tc_tiling_on_sc=False),
        scratch_types=[
            pltpu.VMEM((BLK, CHUNK), jnp.int32),
            pltpu.VMEM((BLK, CHUNK), jnp.int32),
            pltpu.VMEM((CHUNK, HD), jnp.float32),
            pltpu.VMEM((CHUNK, HD), jnp.float32),
            pltpu.VMEM((CHUNK, HD), jnp.float32),
            pltpu.VMEM((CHUNK, HD), jnp.float32),
            pltpu.VMEM((CHUNK, HD), jnp.float32),
            pltpu.VMEM_SHARED((NP, HD), jnp.float32),
            pltpu.VMEM_SHARED((NP, HD), jnp.float32),
            pltpu.SemaphoreType.DMA,
            pltpu.SemaphoreType.DMA,
            pltpu.SemaphoreType.DMA,
            pltpu.SemaphoreType.DMA,
            pltpu.SemaphoreType.DMA,
            pltpu.SemaphoreType.DMA,
            pltpu.SemaphoreType.DMA,
            pltpu.SemaphoreType.DMA,
        ],
    )
    return deg, scatter


def _prep_body(x_ref, dega_ref, degb_ref, w_ref, l_ref, r_ref):
    deg = dega_ref[...] + degb_ref[...]
    dinv = lax.rsqrt(deg + 1.0)
    hs = (
        jnp.dot(x_ref[...], w_ref[...], preferred_element_type=jnp.float32)
        * dinv
    )
    l_ref[...] = hs[:, :HD]
    r_ref[...] = hs[:, HD:]


_prep = pl.pallas_call(
    _prep_body,
    out_shape=(
        jax.ShapeDtypeStruct((NP, HD), jnp.float32),
        jax.ShapeDtypeStruct((NP, HD), jnp.float32),
    ),
)


def _bn_lrelu_half(acc, dinv, b, gm, bt):
    z = dinv * acc + b
    zm = z[:N]
    mean = jnp.mean(zm, axis=0, keepdims=True)
    dev = zm - mean
    var = jnp.mean(dev * dev, axis=0, keepdims=True)
    h = dev * lax.rsqrt(var + 1e-5) * gm + bt
    return jnp.where(h >= 0, h, 0.01 * h)


def _bnhalf_body(a0_ref, a1_ref, hs_ref, dega_ref, degb_ref,
                 b_ref, g_ref, bt_ref, h_ref):
    deg = dega_ref[...] + degb_ref[...]
    dinv = lax.rsqrt(deg + 1.0)
    h_ref[...] = _bn_lrelu_half(a0_ref[...] + a1_ref[...] + hs_ref[...],
                                dinv, b_ref[...], g_ref[...], bt_ref[...])


_bnhalf = pl.pallas_call(
    _bnhalf_body, out_shape=jax.ShapeDtypeStruct((N, HD), jnp.float32)
)


def _mmpart_body(hl_ref, hr_ref, dega_ref, degb_ref, w_ref, l_ref, r_ref):
    deg = dega_ref[...] + degb_ref[...]
    dinv = lax.rsqrt(deg + 1.0)
    hl = hl_ref[...]
    hr = hr_ref[...]
    w = w_ref[...]
    dv = dinv[:N]
    zpad = jnp.zeros((NP - N, HD), jnp.float32)
    l_ref[pl.ds(0, N), :] = (
        jnp.dot(hl, w[:HD, :HD], preferred_element_type=jnp.float32)
        + jnp.dot(hr, w[HD:, :HD], preferred_element_type=jnp.float32)
    ) * dv
    l_ref[pl.ds(N, NP - N), :] = zpad
    r_ref[pl.ds(0, N), :] = (
        jnp.dot(hl, w[:HD, HD:], preferred_element_type=jnp.float32)
        + jnp.dot(hr, w[HD:, HD:], preferred_element_type=jnp.float32)
    ) * dv
    r_ref[pl.ds(N, NP - N), :] = zpad


_mmpart = pl.pallas_call(
    _mmpart_body,
    out_shape=(
        jax.ShapeDtypeStruct((NP, HD), jnp.float32),
        jax.ShapeDtypeStruct((NP, HD), jnp.float32),
    ),
)


def _pool_body(hl_ref, hr_ref, batch_ref, lw_ref, lb_ref, out_ref):
    brow = batch_ref[...][:, :N]
    oh = (lax.broadcasted_iota(jnp.int32, (G, N), 0) == brow).astype(
        jnp.float32
    )
    sums_l = jnp.dot(oh, hl_ref[...], preferred_element_type=jnp.float32)
    sums_r = jnp.dot(oh, hr_ref[...], preferred_element_type=jnp.float32)
    counts = jnp.sum(oh, axis=1, keepdims=True)
    cinv = 1.0 / jnp.maximum(counts, 1.0)
    lw = lw_ref[...]
    out_ref[...] = (
        jnp.dot(sums_l * cinv, lw[:HD], preferred_element_type=jnp.float32)
        + jnp.dot(sums_r * cinv, lw[HD:], preferred_element_type=jnp.float32)
        + lb_ref[...]
    )


_pool = pl.pallas_call(
    _pool_body, out_shape=jax.ShapeDtypeStruct((G, 1), jnp.float32)
)


def kernel(x, edge_index, batch, W1, b1, g1, bt1, W2, b2, g2, bt2,
           W3, b3, g3, bt3, lw, lb):
    src = edge_index[0]
    dst = edge_index[1]
    pad = jnp.full((EP - E,), N, jnp.int32)
    srcp = jnp.concatenate([src, pad]).reshape(NW, CPT, CHUNK)
    dstp = jnp.concatenate([dst, pad]).reshape(NW, CPT, CHUNK)
    x_pad = jnp.pad(x, ((0, NP - N), (0, 0)))
    batch_row = jnp.pad(batch, (0, NP - N), constant_values=G).reshape(1, NP)

    _deg_kernel, _scatter_kernel = _sc_kernels()
    degs = _deg_kernel(dstp)
    dega = degs[0].reshape(NP, 1)
    degb = degs[1].reshape(NP, 1)

    b1r, g1r, bt1r = b1.reshape(1, D), g1.reshape(1, D), bt1.reshape(1, D)
    b2r, g2r, bt2r = b2.reshape(1, D), g2.reshape(1, D), bt2.reshape(1, D)
    b3r, g3r, bt3r = b3.reshape(1, D), g3.reshape(1, D), bt3.reshape(1, D)

    hsl, hsr = _prep(x_pad, dega, degb, W1)
    for (br, gr, btr, Wn) in ((b1r, g1r, bt1r, W2), (b2r, g2r, bt2r, W3)):
        acc = _scatter_kernel(hsl, hsr, srcp, dstp)
        hl = _bnhalf(acc[0, 0], acc[1, 0], hsl, dega, degb,
                     br[:, :HD], gr[:, :HD], btr[:, :HD])
        hr = _bnhalf(acc[0, 1], acc[1, 1], hsr, dega, degb,
                     br[:, HD:], gr[:, HD:], btr[:, HD:])
        hsl, hsr = _mmpart(hl, hr, dega, degb, Wn)
    acc = _scatter_kernel(hsl, hsr, srcp, dstp)
    hl = _bnhalf(acc[0, 0], acc[1, 0], hsl, dega, degb,
                 b3r[:, :HD], g3r[:, :HD], bt3r[:, :HD])
    hr = _bnhalf(acc[0, 1], acc[1, 1], hsr, dega, degb,
                 b3r[:, HD:], g3r[:, HD:], bt3r[:, HD:])
    return _pool(hl, hr, batch_row, lw, lb.reshape(1, 1))


# speedup vs baseline: 2.1686x; 1.0868x over previous
import functools

import jax
import jax.numpy as jnp
from jax import lax
from jax.experimental import pallas as pl
from jax.experimental.pallas import tpu as pltpu
from jax.experimental.pallas import tpu_sc as plsc

N = 10000
E = 320000
D = 128
G = 16
NP = 10240
NC = 2
NS = 16
NW = NC * NS
HD = 64
CHUNK = 128
CPT = 80
BLK = 16
NB = CPT // BLK
DBLK = 8
DNB = CPT // DBLK
EPT = CPT * CHUNK
EP = NW * EPT
RPT = NP // NS

def _deg_body(dst_hbm, out_hbm, dstb, ones_v, zbuf, degs):
    c = lax.axis_index("c")
    s = lax.axis_index("s")
    gid = c * NS + s
    ones16 = jnp.ones((16,), jnp.float32)
    for k in range(CHUNK // 16):
        ones_v[pl.ds(k * 16, 16)] = ones16

    @pl.when(s == 0)
    def _():
        zeros16 = jnp.zeros((16,), jnp.float32)
        zn = zbuf.shape[0]

        def zbody(i, carry):
            zbuf[pl.ds(i * 16, 16)] = zeros16
            return carry

        lax.fori_loop(0, zn // 16, zbody, 0)
        for i in range(NP // zn):
            pltpu.sync_copy(zbuf, degs.at[pl.ds(i * zn, zn)])

    plsc.subcore_barrier()

    def bbody(b, carry):
        pltpu.sync_copy(dst_hbm.at[gid].at[pl.ds(b * DBLK, DBLK)], dstb)

        def abody(j, carry2):
            pltpu.sync_copy(ones_v, degs.at[dstb.at[j]], add=True)
            return carry2

        lax.fori_loop(0, DBLK, abody, 0)
        return carry

    lax.fori_loop(0, DNB, bbody, 0)
    plsc.subcore_barrier()

    @pl.when(s == 0)
    def _():
        zn = zbuf.shape[0]
        for i in range(NP // zn):
            pltpu.sync_copy(degs.at[pl.ds(i * zn, zn)], zbuf)
            pltpu.sync_copy(zbuf, out_hbm.at[c].at[pl.ds(i * zn, zn)])


def _scatter_body(hsl_hbm, hsr_hbm, src_hbm, dst_hbm, out_hbm,
                  srcb, dstb, buf0, buf1, buf2, buf3, zf, hsm, accs,
                  g0, g1, g2, g3, s0, s1, s2, s3):
    c = lax.axis_index("c")
    s = lax.axis_index("s")
    gid = c * NS + s
    bufs = (buf0, buf1, buf2, buf3)
    gsems = (g0, g1, g2, g3)
    ssems = (s0, s1, s2, s3)
    base = s * RPT

    zeros16 = jnp.zeros((16,), jnp.float32)

    def zbody(r, carry):
        for k in range(HD // 16):
            zf[r, pl.ds(k * 16, 16)] = zeros16
        return carry

    lax.fori_loop(0, CHUNK, zbody, 0)

    for h, hs_hbm in enumerate((hsl_hbm, hsr_hbm)):
        pltpu.sync_copy(hs_hbm.at[pl.ds(base, RPT)], hsm.at[pl.ds(base, RPT)])
        for i in range(RPT // CHUNK):
            pltpu.sync_copy(zf, accs.at[pl.ds(base + i * CHUNK, CHUNK)])
        plsc.subcore_barrier()

        def bbody(b, carry):
            pltpu.sync_copy(src_hbm.at[gid].at[pl.ds(b * BLK, BLK)], srcb)
            pltpu.sync_copy(dst_hbm.at[gid].at[pl.ds(b * BLK, BLK)], dstb)
            gd = [None] * BLK
            sd = [None] * BLK
            for j in range(BLK):
                k = j % 4
                if j >= 4:
                    sd[j - 4].wait()
                gd[j] = pltpu.async_copy(hsm.at[srcb.at[j]], bufs[k], gsems[k])
                if j >= 2:
                    p = j - 2
                    gd[p].wait()
                    sd[p] = pltpu.async_copy(
                        bufs[p % 4], accs.at[dstb.at[p]], ssems[p % 4],
                        add=True,
                    )
            for p in (BLK - 2, BLK - 1):
                gd[p].wait()
                sd[p] = pltpu.async_copy(
                    bufs[p % 4], accs.at[dstb.at[p]], ssems[p % 4], add=True
                )
            for p in range(BLK - 4, BLK):
                sd[p].wait()
            return carry

        lax.fori_loop(0, NB, bbody, 0)
        plsc.subcore_barrier()

        for i in range(RPT // CHUNK):
            pltpu.sync_copy(accs.at[pl.ds(base + i * CHUNK, CHUNK)], buf0)
            pltpu.sync_copy(
                buf0, out_hbm.at[c].at[h].at[pl.ds(base + i * CHUNK, CHUNK)]
            )
        plsc.subcore_barrier()


@functools.cache
def _sc_kernels():
    mesh = plsc.VectorSubcoreMesh(
        core_axis_name="c", subcore_axis_name="s",
        num_cores=NC, num_subcores=NS,
    )
    deg = pl.kernel(
        _deg_body,
        out_type=jax.ShapeDtypeStruct((NC, NP), jnp.float32),
        mesh=mesh,
        scratch_types=[
            pltpu.VMEM((DBLK, CHUNK), jnp.int32),
            pltpu.VMEM((CHUNK,), jnp.float32),
            pltpu.VMEM((2048,), jnp.float32),
            pltpu.VMEM_SHARED((NP,), jnp.float32),
        ],
    )
    scatter = pl.kernel(
        _scatter_body,
        out_type=jax.ShapeDtypeStruct((NC, 2, NP, HD), jnp.float32),
        mesh=mesh,
        compiler_params=pltpu.CompilerParams(use_tc_tiling_on_sc=False),
        scratch_types=[
            pltpu.VMEM((BLK, CHUNK), jnp.int32),
            pltpu.VMEM((BLK, CHUNK), jnp.int32),
            pltpu.VMEM((CHUNK, HD), jnp.float32),
            pltpu.VMEM((CHUNK, HD), jnp.float32),
            pltpu.VMEM((CHUNK, HD), jnp.float32),
            pltpu.VMEM((CHUNK, HD), jnp.float32),
            pltpu.VMEM((CHUNK, HD), jnp.float32),
            pltpu.VMEM_SHARED((NP, HD), jnp.float32),
            pltpu.VMEM_SHARED((NP, HD), jnp.float32),
            pltpu.SemaphoreType.DMA,
            pltpu.SemaphoreType.DMA,
            pltpu.SemaphoreType.DMA,
            pltpu.SemaphoreType.DMA,
            pltpu.SemaphoreType.DMA,
            pltpu.SemaphoreType.DMA,
            pltpu.SemaphoreType.DMA,
            pltpu.SemaphoreType.DMA,
        ],
    )
    return deg, scatter


def _prep_body(x_ref, dega_ref, degb_ref, w_ref, l_ref, r_ref):
    deg = dega_ref[...] + degb_ref[...]
    dinv = lax.rsqrt(deg + 1.0)
    hs = (
        jnp.dot(x_ref[...], w_ref[...], preferred_element_type=jnp.float32)
        * dinv
    )
    l_ref[...] = hs[:, :HD]
    r_ref[...] = hs[:, HD:]


_prep = pl.pallas_call(
    _prep_body,
    out_shape=(
        jax.ShapeDtypeStruct((NP, HD), jnp.float32),
        jax.ShapeDtypeStruct((NP, HD), jnp.float32),
    ),
)


def _bn_lrelu_half(acc, dinv, b, gm, bt):
    z = dinv * acc + b
    zm = z[:N]
    mean = jnp.mean(zm, axis=0, keepdims=True)
    dev = zm - mean
    var = jnp.mean(dev * dev, axis=0, keepdims=True)
    h = dev * lax.rsqrt(var + 1e-5) * gm + bt
    return jnp.where(h >= 0, h, 0.01 * h)


def _bnhalf_body(a0_ref, a1_ref, hs_ref, dega_ref, degb_ref,
                 b_ref, g_ref, bt_ref, h_ref):
    deg = dega_ref[...] + degb_ref[...]
    dinv = lax.rsqrt(deg + 1.0)
    h_ref[...] = _bn_lrelu_half(a0_ref[...] + a1_ref[...] + hs_ref[...],
                                dinv, b_ref[...], g_ref[...], bt_ref[...])


_bnhalf = pl.pallas_call(
    _bnhalf_body, out_shape=jax.ShapeDtypeStruct((N, HD), jnp.float32)
)


def _mmpart_body(hl_ref, hr_ref, dega_ref, degb_ref, w_ref, l_ref, r_ref):
    deg = dega_ref[...] + degb_ref[...]
    dinv = lax.rsqrt(deg + 1.0)
    hl = hl_ref[...]
    hr = hr_ref[...]
    w = w_ref[...]
    dv = dinv[:N]
    zpad = jnp.zeros((NP - N, HD), jnp.float32)
    l_ref[pl.ds(0, N), :] = (
        jnp.dot(hl, w[:HD, :HD], preferred_element_type=jnp.float32)
        + jnp.dot(hr, w[HD:, :HD], preferred_element_type=jnp.float32)
    ) * dv
    l_ref[pl.ds(N, NP - N), :] = zpad
    r_ref[pl.ds(0, N), :] = (
        jnp.dot(hl, w[:HD, HD:], preferred_element_type=jnp.float32)
        + jnp.dot(hr, w[HD:, HD:], preferred_element_type=jnp.float32)
    ) * dv
    r_ref[pl.ds(N, NP - N), :] = zpad


_mmpart = pl.pallas_call(
    _mmpart_body,
    out_shape=(
        jax.ShapeDtypeStruct((NP, HD), jnp.float32),
        jax.ShapeDtypeStruct((NP, HD), jnp.float32),
    ),
)


def _pool_body(hl_ref, hr_ref, batch_ref, lw_ref, lb_ref, out_ref):
    brow = batch_ref[...][:, :N]
    oh = (lax.broadcasted_iota(jnp.int32, (G, N), 0) == brow).astype(
        jnp.float32
    )
    sums_l = jnp.dot(oh, hl_ref[...], preferred_element_type=jnp.float32)
    sums_r = jnp.dot(oh, hr_ref[...], preferred_element_type=jnp.float32)
    counts = jnp.sum(oh, axis=1, keepdims=True)
    cinv = 1.0 / jnp.maximum(counts, 1.0)
    lw = lw_ref[...]
    out_ref[...] = (
        jnp.dot(sums_l * cinv, lw[:HD], preferred_element_type=jnp.float32)
        + jnp.dot(sums_r * cinv, lw[HD:], preferred_element_type=jnp.float32)
        + lb_ref[...]
    )


_pool = pl.pallas_call(
    _pool_body, out_shape=jax.ShapeDtypeStruct((G, 1), jnp.float32)
)


def kernel(x, edge_index, batch, W1, b1, g1, bt1, W2, b2, g2, bt2,
           W3, b3, g3, bt3, lw, lb):
    src = edge_index[0]
    dst = edge_index[1]
    pad = jnp.full((EP - E,), N, jnp.int32)
    srcp = jnp.concatenate([src, pad]).reshape(NW, CPT, CHUNK)
    dstp = jnp.concatenate([dst, pad]).reshape(NW, CPT, CHUNK)
    x_pad = jnp.pad(x, ((0, NP - N), (0, 0)))
    batch_row = jnp.pad(batch, (0, NP - N), constant_values=G).reshape(1, NP)

    _deg_kernel, _scatter_kernel = _sc_kernels()
    degs = _deg_kernel(dstp)
    dega = degs[0].reshape(NP, 1)
    degb = degs[1].reshape(NP, 1)

    b1r, g1r, bt1r = b1.reshape(1, D), g1.reshape(1, D), bt1.reshape(1, D)
    b2r, g2r, bt2r = b2.reshape(1, D), g2.reshape(1, D), bt2.reshape(1, D)
    b3r, g3r, bt3r = b3.reshape(1, D), g3.reshape(1, D), bt3.reshape(1, D)

    hsl, hsr = _prep(x_pad, dega, degb, W1)
    for (br, gr, btr, Wn) in ((b1r, g1r, bt1r, W2), (b2r, g2r, bt2r, W3)):
        acc = _scatter_kernel(hsl, hsr, srcp, dstp)
        hl = _bnhalf(acc[0, 0], acc[1, 0], hsl, dega, degb,
                     br[:, :HD], gr[:, :HD], btr[:, :HD])
        hr = _bnhalf(acc[0, 1], acc[1, 1], hsr, dega, degb,
                     br[:, HD:], gr[:, HD:], btr[:, HD:])
        hsl, hsr = _mmpart(hl, hr, dega, degb, Wn)
    acc = _scatter_kernel(hsl, hsr, srcp, dstp)
    hl = _bnhalf(acc[0, 0], acc[1, 0], hsl, dega, degb,
                 b3r[:, :HD], g3r[:, :HD], bt3r[:, :HD])
    hr = _bnhalf(acc[0, 1], acc[1, 1], hsr, dega, degb,
                 b3r[:, HD:], g3r[:, HD:], bt3r[:, HD:])
    return _pool(hl, hr, batch_row, lw, lb.reshape(1, 1))
